# baseline (speedup 1.0000x reference)
import functools

import jax
import jax.numpy as jnp
from jax import lax
from jax.experimental import pallas as pl
from jax.experimental.pallas import tpu as pltpu
from jax.experimental.pallas import tpu_sc as plsc

DIMS = 8
KC = 512
NG = 4
B = 8
HW = 1024
BPS = 4
SC_CORES = 2
LOGVAR_MIN, LOGVAR_MAX = -30.0, 20.0
KL_SCALE = 1.4426 * 0.5


def _tc_body(prior_ref, z_ref, idx_ref, kl_ref):
    prior = prior_ref[...]
    w0 = jnp.concatenate([0.5 * prior * prior, prior], axis=1)
    part = jnp.float32(0.0)
    for b2 in range(BPS):
        zb = z_ref[b2]
        mu = zb[:NG * DIMS, :]
        lv = jnp.clip(zb[NG * DIMS:, :], LOGVAR_MIN, LOGVAR_MAX)
        iv = jnp.exp(-lv)
        a = 1.0 - iv
        bb = mu * iv
        for g in range(NG):
            fg = jnp.concatenate([a[g * DIMS:(g + 1) * DIMS, :],
                                  bb[g * DIMS:(g + 1) * DIMS, :]], axis=0)
            sg = jax.lax.dot(w0, fg,
                             precision=jax.lax.Precision.HIGHEST)
            am = jnp.argmax(sg, axis=0).astype(jnp.int32)
            idx_ref[b2, g, :] = am
        var = jnp.exp(lv)
        part = part + jnp.sum(mu * mu + var - 1.0 - lv)

    @pl.when(pl.program_id(0) == 0)
    def _init():
        kl_ref[0, 0] = 0.0

    kl_ref[0, 0] += part * jnp.float32(KL_SCALE / (B * NG * HW))


def _tc_stage(prior, z3):
    return pl.pallas_call(
        _tc_body,
        grid=(B // BPS,),
        in_specs=[
            pl.BlockSpec((KC, DIMS), lambda b: (0, 0)),
            pl.BlockSpec((BPS, 2 * NG * DIMS, HW), lambda b: (b, 0, 0)),
        ],
        out_specs=[
            pl.BlockSpec((BPS, NG, HW), lambda b: (b, 0, 0)),
            pl.BlockSpec((1, 1), lambda b: (0, 0),
                         memory_space=pltpu.SMEM),
        ],
        out_shape=[
            jax.ShapeDtypeStruct((B, NG, HW), jnp.int32),
            jax.ShapeDtypeStruct((1, 1), jnp.float32),
        ],
    )(prior, z3)


@functools.partial(
    pl.kernel,
    mesh=plsc.VectorSubcoreMesh(core_axis_name="c", subcore_axis_name="s"),
    compiler_params=pltpu.CompilerParams(needs_layout_passes=False),
    out_type=jax.ShapeDtypeStruct((B, NG * DIMS, HW), jnp.float32),
    scratch_types=[
        pltpu.VMEM((HW,), jnp.int32),
        pltpu.VMEM((DIMS * KC,), jnp.float32),
        pltpu.VMEM((DIMS, HW), jnp.float32),
    ],
)
def _sc_gather(idx_hbm, pt_hbm, out_hbm, idx_v, pt_v, out_v):
    wid = lax.axis_index("s") * SC_CORES + lax.axis_index("c")
    b = wid // NG
    g = wid % NG
    pltpu.sync_copy(pt_hbm, pt_v)
    pltpu.sync_copy(idx_hbm.at[b, g], idx_v)

    def body(j, carry):
        code = idx_v[pl.ds(pl.multiple_of(j * 16, 16), 16)]
        for d in range(DIMS):
            vals = plsc.load_gather(pt_v, [code + (d * KC)])
            out_v[d, pl.ds(pl.multiple_of(j * 16, 16), 16)] = vals
        return carry

    lax.fori_loop(0, HW // 16, body, 0)
    pltpu.sync_copy(out_v, out_hbm.at[b, pl.ds(g * DIMS, DIMS)])


def kernel(z, prior_samples):
    z3 = jnp.zeros((B, 2 * NG * DIMS, HW), jnp.float32) + prior_samples[0, 0]
    idx, kl = _tc_stage(prior_samples, z3)
    zhat3 = _sc_gather(idx, prior_samples.T.reshape(DIMS * KC))

    kl_loss = kl[0, 0]
    indices = idx.reshape(B, NG, 32, 32)
    zhat = zhat3.reshape(B, NG * DIMS, 32, 32)
    return zhat, kl_loss, indices


# speedup vs baseline: 1.0246x; 1.0246x over previous
import functools

import jax
import jax.numpy as jnp
from jax import lax
from jax.experimental import pallas as pl
from jax.experimental.pallas import tpu as pltpu
from jax.experimental.pallas import tpu_sc as plsc

DIMS = 8
KC = 512
NG = 4
B = 8
HW = 1024
BPS = 4
SC_CORES = 2
LOGVAR_MIN, LOGVAR_MAX = -30.0, 20.0
KL_SCALE = 1.4426 * 0.5


def _tc_body(prior_ref, z_ref, idx_ref, kl_ref):
    prior = prior_ref[...]
    w0 = jnp.concatenate([0.5 * prior * prior, prior], axis=1)
    part = jnp.float32(0.0)
    for b2 in range(BPS):
        zb = z_ref[b2]
        mu = zb[:NG * DIMS, :]
        lv = jnp.clip(zb[NG * DIMS:, :], LOGVAR_MIN, LOGVAR_MAX)
        iv = jnp.exp(-lv)
        a = 1.0 - iv
        bb = mu * iv
        for g in range(NG):
            fg = jnp.concatenate([a[g * DIMS:(g + 1) * DIMS, :],
                                  bb[g * DIMS:(g + 1) * DIMS, :]], axis=0)
            sg = jax.lax.dot(w0, fg,
                             precision=jax.lax.Precision.HIGHEST)
            am = jnp.argmax(sg, axis=0).astype(jnp.int32)
            idx_ref[b2, g, :] = am
        var = jnp.exp(lv)
        part = part + jnp.sum(mu * mu + var - 1.0 - lv)

    @pl.when(pl.program_id(0) == 0)
    def _init():
        kl_ref[0, 0] = 0.0

    kl_ref[0, 0] += part * jnp.float32(KL_SCALE / (B * NG * HW))


def _tc_stage(prior, z3):
    return pl.pallas_call(
        _tc_body,
        grid=(B // BPS,),
        in_specs=[
            pl.BlockSpec((KC, DIMS), lambda b: (0, 0)),
            pl.BlockSpec((BPS, 2 * NG * DIMS, HW), lambda b: (b, 0, 0)),
        ],
        out_specs=[
            pl.BlockSpec((BPS, NG, HW), lambda b: (b, 0, 0)),
            pl.BlockSpec((1, 1), lambda b: (0, 0),
                         memory_space=pltpu.SMEM),
        ],
        out_shape=[
            jax.ShapeDtypeStruct((B, NG, HW), jnp.int32),
            jax.ShapeDtypeStruct((1, 1), jnp.float32),
        ],
    )(prior, z3)


@functools.partial(
    pl.kernel,
    mesh=plsc.VectorSubcoreMesh(core_axis_name="c", subcore_axis_name="s"),
    compiler_params=pltpu.CompilerParams(needs_layout_passes=False),
    out_type=jax.ShapeDtypeStruct((B, NG * DIMS, HW), jnp.float32),
    scratch_types=[
        pltpu.VMEM((HW,), jnp.int32),
        pltpu.VMEM((DIMS * KC,), jnp.float32),
        pltpu.VMEM((DIMS, HW), jnp.float32),
        pltpu.SemaphoreType.DMA,
        pltpu.SemaphoreType.DMA,
    ],
)
def _sc_gather(idx_hbm, pt_hbm, out_hbm, idx_v, pt_v, out_v, sem_t, sem_i):
    wid = lax.axis_index("s") * SC_CORES + lax.axis_index("c")
    b = wid // NG
    g = wid % NG
    cp_t = pltpu.async_copy(pt_hbm, pt_v, sem_t)
    cp_i = pltpu.async_copy(idx_hbm.at[b, g], idx_v, sem_i)
    cp_t.wait()
    cp_i.wait()

    def body(j, carry):
        for jj in range(4):
            o = pl.multiple_of(j * 64 + jj * 16, 16)
            code = idx_v[pl.ds(o, 16)]
            for d in range(DIMS):
                vals = plsc.load_gather(pt_v, [code + (d * KC)])
                out_v[d, pl.ds(o, 16)] = vals
        return carry

    lax.fori_loop(0, HW // 64, body, 0)
    pltpu.sync_copy(out_v, out_hbm.at[b, pl.ds(g * DIMS, DIMS)])


def kernel(z, prior_samples):
    z3 = z.reshape(B, 2 * NG * DIMS, HW)
    idx, kl = _tc_stage(prior_samples, z3)
    zhat3 = _sc_gather(idx, prior_samples.T.reshape(DIMS * KC))

    kl_loss = kl[0, 0]
    indices = idx.reshape(B, NG, 32, 32)
    zhat = zhat3.reshape(B, NG * DIMS, 32, 32)
    return zhat, kl_loss, indices
